# baseline (speedup 1.0000x reference)
import jax
import jax.numpy as jnp
from jax import lax
from jax.experimental import pallas as pl
from jax.experimental.pallas import tpu as pltpu
from jax.experimental.pallas import tpu_sc as plsc

PLAYER_DIM = 64
VENUE_DIM = 32
TEAM_DIM = 32
B = 16384

NC = 2
NS = 16
NW = NC * NS
BPW = B // NW
CHUNK = 128
NCHUNK = BPW // CHUNK


def _body(tp, tv, tt,
          batter_i, bowler_i, non_striker_i, venue_i, batting_i, bowling_i,
          batter_o, bowler_o, non_striker_o, venue_o, batting_o, bowling_o,
          idx_v, rows_p0, rows_p1, rows_s0, rows_s1, sem_g, sem_w):
    wid = lax.axis_index("s") * NC + lax.axis_index("c")
    base = wid * BPW

    lookups = [
        (tp, batter_i, batter_o, rows_p0, PLAYER_DIM),
        (tp, bowler_i, bowler_o, rows_p1, PLAYER_DIM),
        (tp, non_striker_i, non_striker_o, rows_p0, PLAYER_DIM),
        (tv, venue_i, venue_o, rows_s0, VENUE_DIM),
        (tt, batting_i, batting_o, rows_s1, TEAM_DIM),
        (tt, bowling_i, bowling_o, rows_s0, TEAM_DIM),
    ]

    for k, (_t, idx, _o, _r, _d) in enumerate(lookups):
        pltpu.sync_copy(idx.at[pl.ds(base, BPW)], idx_v.at[k])

    def fire_gathers(k):
        table, _idx, _out, rows, d = lookups[k]

        def per_c(c, carry):
            for ch in range(NCHUNK):
                pltpu.async_copy(
                    table.at[c].at[idx_v.at[k, pl.ds(ch * CHUNK, CHUNK)]],
                    rows.at[c, pl.ds(ch * CHUNK, CHUNK)], sem_g)
            return carry

        lax.fori_loop(0, d, per_c, 0, unroll=False)

    def drain_gathers(k):
        table, _idx, _out, rows, _d = lookups[k]
        pltpu.make_async_copy(table.at[:, pl.ds(0, BPW)], rows, sem_g).wait()

    fire_gathers(0)
    writes = [None] * 6
    for k in range(6):
        drain_gathers(k)
        if k >= 1:
            writes[k - 1].wait()
        if k + 1 < 6:
            fire_gathers(k + 1)
        _t, _i, out, rows, _d = lookups[k]
        writes[k] = pltpu.async_copy(rows, out.at[:, pl.ds(base, BPW)], sem_w)
    writes[5].wait()


@jax.jit
def _run(tp, tv, tt, batter_i, bowler_i, non_striker_i,
         venue_i, batting_i, bowling_i):
    f32 = jnp.float32
    out_type = (
        jax.ShapeDtypeStruct((PLAYER_DIM, B), f32),
        jax.ShapeDtypeStruct((PLAYER_DIM, B), f32),
        jax.ShapeDtypeStruct((PLAYER_DIM, B), f32),
        jax.ShapeDtypeStruct((VENUE_DIM, B), f32),
        jax.ShapeDtypeStruct((TEAM_DIM, B), f32),
        jax.ShapeDtypeStruct((TEAM_DIM, B), f32),
    )
    mesh = plsc.VectorSubcoreMesh(
        core_axis_name="c", subcore_axis_name="s",
        num_cores=NC, num_subcores=NS)
    kern = pl.kernel(
        _body,
        out_type,
        mesh=mesh,
        compiler_params=pltpu.CompilerParams(use_tc_tiling_on_sc=False),
        scratch_types=[
            pltpu.VMEM((6, BPW), jnp.int32),
            pltpu.VMEM((PLAYER_DIM, BPW), f32),
            pltpu.VMEM((PLAYER_DIM, BPW), f32),
            pltpu.VMEM((VENUE_DIM, BPW), f32),
            pltpu.VMEM((TEAM_DIM, BPW), f32),
            pltpu.SemaphoreType.DMA,
            pltpu.SemaphoreType.DMA,
        ],
    )
    return kern(tp, tv, tt, batter_i, bowler_i,
                non_striker_i, venue_i, batting_i, bowling_i)


def kernel(player_table, venue_table, team_table, batter_idx, bowler_idx,
           non_striker_idx, venue_idx, batting_team_idx, bowling_team_idx):
    outs = _run(player_table.T, venue_table.T, team_table.T,
                batter_idx.astype(jnp.int32), bowler_idx.astype(jnp.int32),
                non_striker_idx.astype(jnp.int32), venue_idx.astype(jnp.int32),
                batting_team_idx.astype(jnp.int32),
                bowling_team_idx.astype(jnp.int32))
    return tuple(o.T for o in outs)


# speedup vs baseline: 7.0549x; 7.0549x over previous
import jax
import jax.numpy as jnp
from jax import lax
from jax.experimental import pallas as pl
from jax.experimental.pallas import tpu as pltpu
from jax.experimental.pallas import tpu_sc as plsc

PLAYER_DIM = 64
VENUE_DIM = 32
TEAM_DIM = 32
NV = 1001
B = 16384

NC = 2
NS = 16
NW = NC * NS
BPW = B // NW
CHUNK = 128
NCHUNK = BPW // CHUNK
G = BPW // 16


def _body(pt, vT, tT,
          batter_i, bowler_i, non_striker_i, venue_i, batting_i, bowling_i,
          batter_o, bowler_o, non_striker_o, venue_o, batting_o, bowling_o,
          idx_v, rows, rowsT, tab, smallT, sem_g, sem_w):
    wid = lax.axis_index("s") * NC + lax.axis_index("c")
    base = wid * BPW

    all_idx = [batter_i, bowler_i, non_striker_i, venue_i, batting_i, bowling_i]
    for k, idx in enumerate(all_idx):
        pltpu.sync_copy(idx.at[pl.ds(base, BPW)], idx_v.at[k])

    def small_lookup(k, out):
        def per_c(c, carry):
            def per_g(g, carry2):
                r_vec = idx_v[k, pl.ds(g * 16, 16)]
                c_vec = jnp.full((16,), c, dtype=jnp.int32)
                vals = plsc.load_gather(tab, [c_vec, r_vec])
                smallT[c, pl.ds(g * 16, 16)] = vals
                return carry2
            return lax.fori_loop(0, G, per_g, carry)
        lax.fori_loop(0, VENUE_DIM, per_c, 0)
        pltpu.sync_copy(smallT, out.at[:, pl.ds(base, BPW)])

    pltpu.sync_copy(vT, tab)
    small_lookup(3, venue_o)
    pltpu.sync_copy(tT, tab)
    small_lookup(4, batting_o)
    small_lookup(5, bowling_o)

    def player_lookup(k, out):
        cps = []
        for ch in range(NCHUNK):
            cps.append(pltpu.async_copy(
                pt.at[idx_v.at[k, pl.ds(ch * CHUNK, CHUNK)]],
                rows.at[pl.ds(ch * CHUNK, CHUNK)], sem_g))
        for cp in cps:
            cp.wait()

        def per_c(c, carry):
            def per_g(g, carry2):
                b_vec = lax.iota(jnp.int32, 16) + g * 16
                c_vec = jnp.full((16,), c, dtype=jnp.int32)
                vals = plsc.load_gather(rows, [b_vec, c_vec])
                rowsT[c, pl.ds(g * 16, 16)] = vals
                return carry2
            return lax.fori_loop(0, G, per_g, carry)
        lax.fori_loop(0, PLAYER_DIM, per_c, 0)
        pltpu.sync_copy(rowsT, out.at[:, pl.ds(base, BPW)])

    player_lookup(0, batter_o)
    player_lookup(1, bowler_o)
    player_lookup(2, non_striker_o)


@jax.jit
def _run(pt, vT, tT, batter_i, bowler_i, non_striker_i,
         venue_i, batting_i, bowling_i):
    f32 = jnp.float32
    out_type = (
        jax.ShapeDtypeStruct((PLAYER_DIM, B), f32),
        jax.ShapeDtypeStruct((PLAYER_DIM, B), f32),
        jax.ShapeDtypeStruct((PLAYER_DIM, B), f32),
        jax.ShapeDtypeStruct((VENUE_DIM, B), f32),
        jax.ShapeDtypeStruct((TEAM_DIM, B), f32),
        jax.ShapeDtypeStruct((TEAM_DIM, B), f32),
    )
    mesh = plsc.VectorSubcoreMesh(
        core_axis_name="c", subcore_axis_name="s",
        num_cores=NC, num_subcores=NS)
    kern = pl.kernel(
        _body,
        out_type,
        mesh=mesh,
        compiler_params=pltpu.CompilerParams(
            use_tc_tiling_on_sc=False, needs_layout_passes=False),
        scratch_types=[
            pltpu.VMEM((6, BPW), jnp.int32),
            pltpu.VMEM((BPW, PLAYER_DIM), f32),
            pltpu.VMEM((PLAYER_DIM, BPW), f32),
            pltpu.VMEM((VENUE_DIM, NV), f32),
            pltpu.VMEM((VENUE_DIM, BPW), f32),
            pltpu.SemaphoreType.DMA,
            pltpu.SemaphoreType.DMA,
        ],
    )
    return kern(pt, vT, tT, batter_i, bowler_i,
                non_striker_i, venue_i, batting_i, bowling_i)


def kernel(player_table, venue_table, team_table, batter_idx, bowler_idx,
           non_striker_idx, venue_idx, batting_team_idx, bowling_team_idx):
    outs = _run(player_table, venue_table.T, team_table.T,
                batter_idx.astype(jnp.int32), bowler_idx.astype(jnp.int32),
                non_striker_idx.astype(jnp.int32), venue_idx.astype(jnp.int32),
                batting_team_idx.astype(jnp.int32),
                bowling_team_idx.astype(jnp.int32))
    return tuple(o.T for o in outs)
